# baseline (speedup 1.0000x reference)
import functools

import jax
import jax.numpy as jnp
from jax import lax
from jax.experimental import pallas as pl
from jax.experimental.pallas import tpu as pltpu
from jax.experimental.pallas import tpu_sc as plsc

B = 16384
NCAT = 26
VOCAB = 100000
D = 32
NCONT = 13
NBINS = 8

NC, NS = 2, 16
NW = NC * NS
ROWS = B * NCAT
ROWS_PER_W = ROWS // NW
CHUNK = 128
STEPS = ROWS_PER_W // CHUNK


def _gather_body(table_h, idx_h, out_h, idx_v, buf_v, gsem, wsem):
    wid = lax.axis_index("s") * NC + lax.axis_index("c")
    base = wid * ROWS_PER_W
    pltpu.sync_copy(idx_h.at[wid], idx_v)

    def step(j, carry):
        pltpu.async_copy(table_h.at[idx_v.at[j]], buf_v, gsem).wait()
        pltpu.async_copy(buf_v, out_h.at[pl.ds(base + j * CHUNK, CHUNK)],
                         wsem).wait()
        return carry

    lax.fori_loop(0, STEPS, step, 0)


@functools.cache
def _sc_gather():
    return pl.kernel(
        _gather_body,
        out_type=jax.ShapeDtypeStruct((ROWS, D), jnp.float32),
        mesh=plsc.VectorSubcoreMesh(core_axis_name="c", subcore_axis_name="s",
                                    num_cores=NC, num_subcores=NS),
        compiler_params=pltpu.CompilerParams(use_tc_tiling_on_sc=False),
        scratch_types=[
            pltpu.VMEM((STEPS, CHUNK), jnp.int32),
            pltpu.VMEM((CHUNK, D), jnp.float32),
            pltpu.SemaphoreType.DMA,
            pltpu.SemaphoreType.DMA,
        ],
    )


def _plr_kernel(xrep_ref, bb_ref, w_ref, b_ref, o_ref):
    enc = jnp.maximum(1.0 - jnp.abs(xrep_ref[...] - bb_ref[...]), 0.0)
    o_ref[...] = (
        jnp.dot(enc, w_ref[...], preferred_element_type=jnp.float32)
        + b_ref[...]
    )


def _plr(x_rep, bb_flat, W, b):
    blk = 2048
    return pl.pallas_call(
        _plr_kernel,
        grid=(B // blk,),
        in_specs=[
            pl.BlockSpec((blk, NCONT * NBINS), lambda i: (i, 0)),
            pl.BlockSpec((1, NCONT * NBINS), lambda i: (0, 0)),
            pl.BlockSpec((NCONT * NBINS, NCONT * D), lambda i: (0, 0)),
            pl.BlockSpec((1, NCONT * D), lambda i: (0, 0)),
        ],
        out_specs=pl.BlockSpec((blk, NCONT * D), lambda i: (i, 0)),
        out_shape=jax.ShapeDtypeStruct((B, NCONT * D), jnp.float32),
    )(x_rep, bb_flat, W, b)


def kernel(x_cat, x_cont, tables, bin_boundaries, W, b):
    table_flat = tables.reshape(NCAT * VOCAB, D)
    idx_flat = (x_cat.astype(jnp.int32)
                + (jnp.arange(NCAT, dtype=jnp.int32) * VOCAB)[None, :])
    idx3 = idx_flat.reshape(NW, STEPS, CHUNK)
    cat_flat = _sc_gather()(table_flat, idx3)

    x_rep = jnp.repeat(x_cont, NBINS, axis=1)
    bb_flat = bin_boundaries.reshape(1, NCONT * NBINS)
    cont = _plr(x_rep, bb_flat, W, b.reshape(1, NCONT * D))

    return jnp.concatenate(
        [cat_flat.reshape(B, NCAT, D), cont.reshape(B, NCONT, D)], axis=1)
